# Initial kernel scaffold; baseline (speedup 1.0000x reference)
import jax
import jax.numpy as jnp
from jax.experimental import pallas as pl


def kernel(x, edge_index, W1l, b1l, W1r, W2l, b2l, W2r, W3l, b3l, W3r):
    raise NotImplementedError("write your pallas kernel here")


# speedup vs baseline: 5.5645x; 5.5645x over previous
import functools

import jax
import jax.numpy as jnp
from jax import lax
from jax.experimental import pallas as pl
from jax.experimental.pallas import tpu as pltpu
from jax.experimental.pallas import tpu_sc as plsc

NC = 2
NS = 16
NW = NC * NS
CH = 64
L = 16


def _round_up(a, b):
    return (a + b - 1) // b * b


@functools.lru_cache(maxsize=None)
def _make_sc_scatter(N, D, CPW):
    N_ACC = _round_up(N + 1, NS * 8)
    RPT = N_ACC // NS
    assert CPW >= 3 and CPW % 2 == 1
    nfull, rem = divmod(RPT, CH)

    mesh = plsc.VectorSubcoreMesh(core_axis_name="c", subcore_axis_name="s",
                                  num_cores=NC, num_subcores=NS)

    out_type = jax.ShapeDtypeStruct((NC, N_ACC, D), jnp.float32)
    scratch = [
        pltpu.VMEM_SHARED((N_ACC, D), jnp.float32),
        pltpu.VMEM((2, CH), jnp.int32),
        pltpu.VMEM((2, CH), jnp.int32),
        pltpu.VMEM((CH, D), jnp.float32),
        pltpu.VMEM((CH, D), jnp.float32),
        pltpu.SemaphoreType.DMA,
        pltpu.SemaphoreType.DMA,
        pltpu.SemaphoreType.DMA,
        pltpu.SemaphoreType.DMA,
    ]

    def body(y, idxr, outS, acc, ib0, ib1, rowsA, rowsB,
             semA, semB, semI0, semI1):
        c = lax.axis_index("c")
        s = lax.axis_index("s")
        w = c * NS + s
        base = s * RPT

        z16 = jnp.zeros((L,), jnp.float32)

        def zrow(i, carry):
            for k in range(D // L):
                rowsA[i, pl.ds(L * k, L)] = z16
            return carry
        lax.fori_loop(0, CH, zrow, 0)
        for b in range(nfull):
            pltpu.sync_copy(rowsA, acc.at[pl.ds(base + b * CH, CH)])
        if rem:
            pltpu.sync_copy(rowsA.at[pl.ds(0, rem)],
                            acc.at[pl.ds(base + nfull * CH, rem)])

        plsc.subcore_barrier()

        pltpu.sync_copy(idxr.at[w, 0], ib0)
        pltpu.async_copy(idxr.at[w, 1], ib1, semI1)
        pltpu.async_copy(y.at[ib0.at[0]], rowsA, semA)

        def pair(jj, carry):
            j0 = 2 * jj
            pltpu.make_async_copy(y.at[ib0.at[0]], rowsA, semA).wait()
            pltpu.make_async_copy(idxr.at[w, 0], ib1, semI1).wait()
            pltpu.async_copy(y.at[ib1.at[0]], rowsB, semB)
            pltpu.sync_copy(rowsA, acc.at[ib0.at[1]], add=True)
            pltpu.async_copy(idxr.at[w, j0 + 2], ib0, semI0)
            pltpu.make_async_copy(y.at[ib1.at[0]], rowsB, semB).wait()
            pltpu.make_async_copy(idxr.at[w, 0], ib0, semI0).wait()
            pltpu.async_copy(y.at[ib0.at[0]], rowsA, semA)
            pltpu.sync_copy(rowsB, acc.at[ib1.at[1]], add=True)
            pltpu.async_copy(idxr.at[w, j0 + 3], ib1, semI1)
            return carry
        lax.fori_loop(0, (CPW - 1) // 2, pair, 0)

        pltpu.make_async_copy(y.at[ib0.at[0]], rowsA, semA).wait()
        pltpu.sync_copy(rowsA, acc.at[ib0.at[1]], add=True)
        pltpu.make_async_copy(idxr.at[w, 0], ib1, semI1).wait()

        plsc.subcore_barrier()

        bufs = (rowsA, rowsB)
        sems = (semA, semB)
        pieces = [(b * CH, CH) for b in range(nfull)]
        if rem:
            pieces.append((nfull * CH, rem))
        for i, (off, sz) in enumerate(pieces):
            buf = bufs[i % 2].at[pl.ds(0, sz)]
            sem = sems[i % 2]
            if i >= 2:
                poff, psz = pieces[i - 2]
                pltpu.make_async_copy(
                    bufs[i % 2].at[pl.ds(0, psz)],
                    outS.at[c, pl.ds(base + poff, psz)], sem).wait()
            pltpu.sync_copy(acc.at[pl.ds(base + off, sz)], buf)
            pltpu.async_copy(buf, outS.at[c, pl.ds(base + off, sz)], sem)
        for i in range(max(0, len(pieces) - 2), len(pieces)):
            poff, psz = pieces[i]
            pltpu.make_async_copy(
                bufs[i % 2].at[pl.ds(0, psz)],
                outS.at[c, pl.ds(base + poff, psz)], sems[i % 2]).wait()

    return pl.kernel(body, out_type=out_type, mesh=mesh,
                     scratch_types=scratch)


@functools.lru_cache(maxsize=None)
def _make_sc_count(N, CPW):
    D = 128
    N_ACC = _round_up(N + 1, NS * 8)
    RPT = N_ACC // NS
    nfull, rem = divmod(RPT, CH)

    mesh = plsc.VectorSubcoreMesh(core_axis_name="c", subcore_axis_name="s",
                                  num_cores=NC, num_subcores=NS)

    out_type = jax.ShapeDtypeStruct((NC, N_ACC, D), jnp.float32)
    scratch = [
        pltpu.VMEM_SHARED((N_ACC, D), jnp.float32),
        pltpu.VMEM((CPW + 1, CH), jnp.int32),
        pltpu.VMEM((CH, D), jnp.float32),
        pltpu.VMEM((CH, D), jnp.float32),
        pltpu.SemaphoreType.DMA,
        pltpu.SemaphoreType.DMA,
    ]

    def body(dstr, outC, acc, dstv, rowsA, rowsB, semA, semB):
        c = lax.axis_index("c")
        s = lax.axis_index("s")
        w = c * NS + s
        base = s * RPT

        z16 = jnp.zeros((L,), jnp.float32)
        one16 = jnp.ones((L,), jnp.float32)

        def zrow(i, carry):
            for k in range(D // L):
                rowsA[i, pl.ds(L * k, L)] = z16
            return carry
        lax.fori_loop(0, CH, zrow, 0)
        for b in range(nfull):
            pltpu.sync_copy(rowsA, acc.at[pl.ds(base + b * CH, CH)])
        if rem:
            pltpu.sync_copy(rowsA.at[pl.ds(0, rem)],
                            acc.at[pl.ds(base + nfull * CH, rem)])

        def orow(i, carry):
            for k in range(D // L):
                rowsA[i, pl.ds(L * k, L)] = one16
            return carry
        lax.fori_loop(0, CH, orow, 0)

        pltpu.sync_copy(dstr.at[w], dstv)

        plsc.subcore_barrier()

        def step(j, carry):
            pltpu.sync_copy(rowsA, acc.at[dstv.at[j]], add=True)
            return carry
        lax.fori_loop(0, CPW, step, 0)

        plsc.subcore_barrier()

        bufs = (rowsB, rowsA)
        sems = (semB, semA)
        pieces = [(b * CH, CH) for b in range(nfull)]
        if rem:
            pieces.append((nfull * CH, rem))
        for i, (off, sz) in enumerate(pieces):
            buf = bufs[i % 2].at[pl.ds(0, sz)]
            sem = sems[i % 2]
            if i >= 2:
                poff, psz = pieces[i - 2]
                pltpu.make_async_copy(
                    bufs[i % 2].at[pl.ds(0, psz)],
                    outC.at[c, pl.ds(base + poff, psz)], sem).wait()
            pltpu.sync_copy(acc.at[pl.ds(base + off, sz)], buf)
            pltpu.async_copy(buf, outC.at[c, pl.ds(base + off, sz)], sem)
        for i in range(max(0, len(pieces) - 2), len(pieces)):
            poff, psz = pieces[i]
            pltpu.make_async_copy(
                bufs[i % 2].at[pl.ds(0, psz)],
                outC.at[c, pl.ds(base + poff, psz)], sems[i % 2]).wait()

    return pl.kernel(body, out_type=out_type, mesh=mesh,
                     scratch_types=scratch)


def _mm(a, b):
    return lax.dot_general(a, b, (((1,), (0,)), ((), ())),
                           precision=lax.Precision.HIGHEST,
                           preferred_element_type=jnp.float32)


@functools.lru_cache(maxsize=None)
def _make_tc_in(N, D, BLK):
    grid = (N // BLK,)

    def body(x_ref, wl_ref, b_ref, wr_ref, yl_ref, yr_ref):
        xb = x_ref[...]
        yl_ref[...] = _mm(xb, wl_ref[...])
        yr_ref[...] = _mm(xb, wr_ref[...]) + b_ref[...]

    return pl.pallas_call(
        body,
        grid=grid,
        in_specs=[
            pl.BlockSpec((BLK, D), lambda i: (i, 0)),
            pl.BlockSpec((D, D), lambda i: (0, 0)),
            pl.BlockSpec((1, D), lambda i: (0, 0)),
            pl.BlockSpec((D, D), lambda i: (0, 0)),
        ],
        out_specs=[
            pl.BlockSpec((BLK, D), lambda i: (i, 0)),
            pl.BlockSpec((BLK, D), lambda i: (i, 0)),
        ],
        out_shape=[
            jax.ShapeDtypeStruct((N, D), jnp.float32),
            jax.ShapeDtypeStruct((N, D), jnp.float32),
        ],
    )


def _combine(S_ref, C_ref, yrp_ref):
    agg = S_ref[0] + S_ref[1]
    cnt = C_ref[0, :, 0:1] + C_ref[1, :, 0:1]
    inv = 1.0 / jnp.maximum(cnt, 1.0)
    return agg * inv + yrp_ref[...]


@functools.lru_cache(maxsize=None)
def _make_tc_mid(N, N_ACC, D, BLK):
    grid = (N // BLK,)

    def body(S_ref, C_ref, yrp_ref, wl_ref, b_ref, wr_ref, yl_ref, yr_ref):
        h = jnp.maximum(_combine(S_ref, C_ref, yrp_ref), 0.0)
        yl_ref[...] = _mm(h, wl_ref[...])
        yr_ref[...] = _mm(h, wr_ref[...]) + b_ref[...]

    return pl.pallas_call(
        body,
        grid=grid,
        in_specs=[
            pl.BlockSpec((NC, BLK, D), lambda i: (0, i, 0)),
            pl.BlockSpec((NC, BLK, 128), lambda i: (0, i, 0)),
            pl.BlockSpec((BLK, D), lambda i: (i, 0)),
            pl.BlockSpec((D, D), lambda i: (0, 0)),
            pl.BlockSpec((1, D), lambda i: (0, 0)),
            pl.BlockSpec((D, D), lambda i: (0, 0)),
        ],
        out_specs=[
            pl.BlockSpec((BLK, D), lambda i: (i, 0)),
            pl.BlockSpec((BLK, D), lambda i: (i, 0)),
        ],
        out_shape=[
            jax.ShapeDtypeStruct((N, D), jnp.float32),
            jax.ShapeDtypeStruct((N, D), jnp.float32),
        ],
    )


@functools.lru_cache(maxsize=None)
def _make_tc_out(N, N_ACC, D, BLK):
    grid = (N // BLK,)

    def body(S_ref, C_ref, yrp_ref, out_ref):
        out_ref[...] = _combine(S_ref, C_ref, yrp_ref)

    return pl.pallas_call(
        body,
        grid=grid,
        in_specs=[
            pl.BlockSpec((NC, BLK, D), lambda i: (0, i, 0)),
            pl.BlockSpec((NC, BLK, 128), lambda i: (0, i, 0)),
            pl.BlockSpec((BLK, D), lambda i: (i, 0)),
        ],
        out_specs=pl.BlockSpec((BLK, D), lambda i: (i, 0)),
        out_shape=jax.ShapeDtypeStruct((N, D), jnp.float32),
    )


def kernel(x, edge_index, W1l, b1l, W1r, W2l, b2l, W2r, W3l, b3l, W3r):
    N, D = x.shape
    E = edge_index.shape[1]

    CPW = -(-E // (NW * CH))
    if CPW % 2 == 0:
        CPW += 1
    E_pad = NW * CPW * CH
    N_ACC = _round_up(N + 1, NS * 8)
    BLK = 1000
    assert N % BLK == 0 and N % NS == 0 and D % L == 0

    src = edge_index[0]
    dst = edge_index[1]
    pad = E_pad - E
    srcp = jnp.concatenate(
        [src, jnp.zeros((pad,), jnp.int32)]).reshape(NW, CPW, CH)
    dstp = jnp.concatenate(
        [dst, jnp.full((pad,), N, jnp.int32)]).reshape(NW, CPW, CH)
    idx = jnp.stack([srcp, dstp], axis=2)
    idx = jnp.pad(idx, ((0, 0), (0, 1), (0, 0), (0, 0)),
                  constant_values=N)
    dsts = jnp.pad(dstp, ((0, 0), (0, 1), (0, 0)),
                   constant_values=N)

    sc_scatter = _make_sc_scatter(N, D, CPW)
    sc_count = _make_sc_count(N, CPW)
    tc_in = _make_tc_in(N, D, BLK)
    tc_mid = _make_tc_mid(N, N_ACC, D, BLK)
    tc_out = _make_tc_out(N, N_ACC, D, BLK)

    b1 = b1l.reshape(1, D)
    b2 = b2l.reshape(1, D)
    b3 = b3l.reshape(1, D)

    C = sc_count(dsts)
    yl1, yr1 = tc_in(x, W1l, b1, W1r)
    S1 = sc_scatter(yl1, idx)
    yl2, yr2 = tc_mid(S1, C, yr1, W2l, b2, W2r)
    S2 = sc_scatter(yl2, idx)
    yl3, yr3 = tc_mid(S2, C, yr2, W3l, b3, W3r)
    S3 = sc_scatter(yl3, idx)
    return tc_out(S3, C, yr3)
